# baseline (speedup 1.0000x reference)
import functools
import math

import jax
import jax.numpy as jnp
from jax import lax
from jax.experimental import pallas as pl
from jax.experimental.pallas import tpu as pltpu
from jax.experimental.pallas import tpu_sc as plsc

N_CODES = 8192
DIM = 64
_BM = 512
_BN = 1024
_RB = 64
_BF = 2048
_SC_WORKERS = 32
_GATHER_CHUNK = 128


def _argmax_body(z_ref, emb_ref, idx_ref, en_scr):
    @pl.when((pl.program_id(0) == 0) & (pl.program_id(1) == 0))
    def _():
        e = emb_ref[...]
        nrm = jnp.sqrt(jnp.sum(e * e, axis=1, keepdims=True))
        en_scr[...] = e / jnp.maximum(nrm, 1e-12)

    zb = z_ref[0]
    big = float(N_CODES)
    nrb = _BM // _RB
    lane = lax.broadcasted_iota(jnp.int32, (_RB, 128), 1).astype(jnp.float32)
    run_m = [jnp.full((_RB, 1), -jnp.inf, jnp.float32) for _ in range(nrb)]
    run_a = [jnp.full((_RB, 1), big, jnp.float32) for _ in range(nrb)]
    for c in range(N_CODES // _BN):
        s = lax.dot_general(
            zb, en_scr[pl.ds(c * _BN, _BN), :],
            (((0,), (1,)), ((), ())),
            preferred_element_type=jnp.float32,
        )
        for r in range(nrb):
            sb = s[r * _RB:(r + 1) * _RB, :]
            v = sb[:, 0:128]
            t_id = jnp.zeros((_RB, 128), jnp.float32)
            for t in range(1, _BN // 128):
                st = sb[:, t * 128:(t + 1) * 128]
                g = st > v
                v = jnp.where(g, st, v)
                t_id = jnp.where(g, jnp.float32(t), t_id)
            colv = t_id * 128.0 + lane
            m = jnp.max(v, axis=1, keepdims=True)
            hit = jnp.where(v == m, colv, big)
            a = jnp.min(hit, axis=1, keepdims=True)
            upd = m > run_m[r]
            run_m[r] = jnp.where(upd, m, run_m[r])
            run_a[r] = jnp.where(upd, a + float(c * _BN), run_a[r])
    out = jnp.concatenate(run_a, axis=0)
    idx_ref[...] = out[:, 0].astype(jnp.int32).reshape(1, 1, _BM)


def _compute_indices(z_chan, emb):
    b = z_chan.shape[0]
    nj = z_chan.shape[2] // _BM
    idx3 = pl.pallas_call(
        _argmax_body,
        grid=(b, nj),
        in_specs=[
            pl.BlockSpec((1, DIM, _BM), lambda i, j: (i, 0, j)),
            pl.BlockSpec((N_CODES, DIM), lambda i, j: (0, 0)),
        ],
        out_specs=pl.BlockSpec((1, 1, _BM), lambda i, j: (i * nj + j, 0, 0)),
        out_shape=jax.ShapeDtypeStruct((b * nj, 1, _BM), jnp.int32),
        scratch_shapes=[pltpu.VMEM((N_CODES, DIM), jnp.float32)],
    )(z_chan, emb)
    return idx3.reshape(-1)


def _sc_gather(emb, idx):
    m = idx.shape[0]
    bpw = m // _SC_WORKERS
    mesh = plsc.VectorSubcoreMesh(core_axis_name="c", subcore_axis_name="s")

    @functools.partial(
        pl.kernel,
        mesh=mesh,
        out_type=jax.ShapeDtypeStruct((m, DIM), jnp.float32),
        compiler_params=pltpu.CompilerParams(use_tc_tiling_on_sc=False),
        scratch_types=[
            pltpu.VMEM((bpw,), jnp.int32),
            pltpu.VMEM((bpw, DIM), jnp.float32),
            pltpu.SemaphoreType.DMA,
        ],
    )
    def gather(table_hbm, idx_hbm, out_hbm, idx_v, rows_v, sem):
        wid = lax.axis_index("s") * 2 + lax.axis_index("c")
        base = wid * bpw
        pltpu.sync_copy(idx_hbm.at[pl.ds(base, bpw)], idx_v)
        copies = [
            pltpu.async_copy(
                table_hbm.at[idx_v.at[pl.ds(j, _GATHER_CHUNK)]],
                rows_v.at[pl.ds(j, _GATHER_CHUNK), :],
                sem,
            )
            for j in range(0, bpw, _GATHER_CHUNK)
        ]
        for cp in copies:
            cp.wait()
        pltpu.sync_copy(rows_v, out_hbm.at[pl.ds(base, bpw)])

    return gather(emb, idx)


def _finalize_body(zq_ref, zqn_ref, bits_ref):
    zq = zq_ref[...]
    qn = jnp.maximum(jnp.sqrt(jnp.sum(zq * zq, axis=1, keepdims=True)), 1e-12)
    zqn = zq / qn
    zqn_ref[...] = zqn
    bits_ref[...] = (zqn * math.sqrt(32.0)).astype(jnp.int32) + 4


def _finalize(zq_raw):
    m = zq_raw.shape[0]
    nb = m // _BF
    return pl.pallas_call(
        _finalize_body,
        grid=(nb,),
        in_specs=[pl.BlockSpec((_BF, DIM), lambda i: (i, 0))],
        out_specs=[
            pl.BlockSpec((_BF, DIM), lambda i: (i, 0)),
            pl.BlockSpec((_BF, DIM), lambda i: (i, 0)),
        ],
        out_shape=[
            jax.ShapeDtypeStruct((m, DIM), jnp.float32),
            jax.ShapeDtypeStruct((m, DIM), jnp.int32),
        ],
    )(zq_raw)


def kernel(z, embedding_weight):
    b, d, h, w = z.shape
    z_chan = z.reshape(b, d, h * w)
    idx = _compute_indices(z_chan, embedding_weight)
    zq_raw = _sc_gather(embedding_weight, idx)
    zqn, bits = _finalize(zq_raw)
    z_q_out = jnp.transpose(zqn.reshape(b, h, w, d), (0, 3, 1, 2))
    bit_indices = bits.reshape(b, h, w, d)
    loss = jnp.zeros((1,), dtype=z.dtype)
    return (z_q_out, idx, bit_indices, loss)


# speedup vs baseline: 1.0246x; 1.0246x over previous
import functools
import math

import jax
import jax.numpy as jnp
from jax import lax
from jax.experimental import pallas as pl
from jax.experimental.pallas import tpu as pltpu
from jax.experimental.pallas import tpu_sc as plsc

N_CODES = 8192
DIM = 64
_BM = 512
_BN = 4096
_RB = 64
_BF = 2048
_SC_WORKERS = 32
_GATHER_CHUNK = 128


def _argmax_body(z_ref, emb_ref, idx_ref, en_scr):
    @pl.when((pl.program_id(0) == 0) & (pl.program_id(1) == 0))
    def _():
        e = emb_ref[...]
        nrm = jnp.sqrt(jnp.sum(e * e, axis=1, keepdims=True))
        en_scr[...] = e / jnp.maximum(nrm, 1e-12)

    zb = z_ref[0]
    big = float(N_CODES)
    nrb = _BM // _RB
    lane = lax.broadcasted_iota(jnp.int32, (_RB, 128), 1).astype(jnp.float32)
    run_m = [jnp.full((_RB, 1), -jnp.inf, jnp.float32) for _ in range(nrb)]
    run_a = [jnp.full((_RB, 1), big, jnp.float32) for _ in range(nrb)]
    for c in range(N_CODES // _BN):
        s = lax.dot_general(
            zb, en_scr[pl.ds(c * _BN, _BN), :],
            (((0,), (1,)), ((), ())),
            preferred_element_type=jnp.float32,
        )
        for r in range(nrb):
            sb = s[r * _RB:(r + 1) * _RB, :]
            v = sb[:, 0:128]
            t_id = jnp.zeros((_RB, 128), jnp.float32)
            for t in range(1, _BN // 128):
                st = sb[:, t * 128:(t + 1) * 128]
                g = st > v
                v = jnp.where(g, st, v)
                t_id = jnp.where(g, jnp.float32(t), t_id)
            colv = t_id * 128.0 + lane
            m = jnp.max(v, axis=1, keepdims=True)
            hit = jnp.where(v == m, colv, big)
            a = jnp.min(hit, axis=1, keepdims=True)
            upd = m > run_m[r]
            run_m[r] = jnp.where(upd, m, run_m[r])
            run_a[r] = jnp.where(upd, a + float(c * _BN), run_a[r])
    out = jnp.concatenate(run_a, axis=0)
    idx_ref[...] = out[:, 0].astype(jnp.int32).reshape(1, 1, _BM)


def _compute_indices(z_chan, emb):
    b = z_chan.shape[0]
    nj = z_chan.shape[2] // _BM
    idx3 = pl.pallas_call(
        _argmax_body,
        grid=(b, nj),
        in_specs=[
            pl.BlockSpec((1, DIM, _BM), lambda i, j: (i, 0, j)),
            pl.BlockSpec((N_CODES, DIM), lambda i, j: (0, 0)),
        ],
        out_specs=pl.BlockSpec((1, 1, _BM), lambda i, j: (i * nj + j, 0, 0)),
        out_shape=jax.ShapeDtypeStruct((b * nj, 1, _BM), jnp.int32),
        scratch_shapes=[pltpu.VMEM((N_CODES, DIM), jnp.float32)],
    )(z_chan, emb)
    return idx3.reshape(-1)


def _sc_gather(emb, idx):
    m = idx.shape[0]
    bpw = m // _SC_WORKERS
    mesh = plsc.VectorSubcoreMesh(core_axis_name="c", subcore_axis_name="s")

    @functools.partial(
        pl.kernel,
        mesh=mesh,
        out_type=jax.ShapeDtypeStruct((m, DIM), jnp.float32),
        compiler_params=pltpu.CompilerParams(use_tc_tiling_on_sc=False),
        scratch_types=[
            pltpu.VMEM((bpw,), jnp.int32),
            pltpu.VMEM((bpw, DIM), jnp.float32),
            pltpu.SemaphoreType.DMA,
        ],
    )
    def gather(table_hbm, idx_hbm, out_hbm, idx_v, rows_v, sem):
        wid = lax.axis_index("s") * 2 + lax.axis_index("c")
        base = wid * bpw
        pltpu.sync_copy(idx_hbm.at[pl.ds(base, bpw)], idx_v)
        copies = [
            pltpu.async_copy(
                table_hbm.at[idx_v.at[pl.ds(j, _GATHER_CHUNK)]],
                rows_v.at[pl.ds(j, _GATHER_CHUNK), :],
                sem,
            )
            for j in range(0, bpw, _GATHER_CHUNK)
        ]
        for cp in copies:
            cp.wait()
        pltpu.sync_copy(rows_v, out_hbm.at[pl.ds(base, bpw)])

    return gather(emb, idx)


def _finalize_body(zq_ref, zqn_ref, bits_ref):
    zq = zq_ref[...]
    qn = jnp.maximum(jnp.sqrt(jnp.sum(zq * zq, axis=1, keepdims=True)), 1e-12)
    zqn = zq / qn
    zqn_ref[...] = zqn
    bits_ref[...] = (zqn * math.sqrt(32.0)).astype(jnp.int32) + 4


def _finalize(zq_raw):
    m = zq_raw.shape[0]
    nb = m // _BF
    return pl.pallas_call(
        _finalize_body,
        grid=(nb,),
        in_specs=[pl.BlockSpec((_BF, DIM), lambda i: (i, 0))],
        out_specs=[
            pl.BlockSpec((_BF, DIM), lambda i: (i, 0)),
            pl.BlockSpec((_BF, DIM), lambda i: (i, 0)),
        ],
        out_shape=[
            jax.ShapeDtypeStruct((m, DIM), jnp.float32),
            jax.ShapeDtypeStruct((m, DIM), jnp.int32),
        ],
    )(zq_raw)


def kernel(z, embedding_weight):
    b, d, h, w = z.shape
    z_chan = z.reshape(b, d, h * w)
    idx = _compute_indices(z_chan, embedding_weight)
    zq_raw = _sc_gather(embedding_weight, idx)
    zqn, bits = _finalize(zq_raw)
    z_q_out = jnp.transpose(zqn.reshape(b, h, w, d), (0, 3, 1, 2))
    bit_indices = bits.reshape(b, h, w, d)
    loss = jnp.zeros((1,), dtype=z.dtype)
    return (z_q_out, idx, bit_indices, loss)


# speedup vs baseline: 1.0267x; 1.0020x over previous
import functools
import math

import jax
import jax.numpy as jnp
from jax import lax
from jax.experimental import pallas as pl
from jax.experimental.pallas import tpu as pltpu
from jax.experimental.pallas import tpu_sc as plsc

N_CODES = 8192
DIM = 64
_BM = 512
_BN = 8192
_RB = 64
_BF = 2048
_SC_WORKERS = 32
_GATHER_CHUNK = 128


def _argmax_body(z_ref, emb_ref, idx_ref, en_scr):
    @pl.when((pl.program_id(0) == 0) & (pl.program_id(1) == 0))
    def _():
        e = emb_ref[...]
        nrm = jnp.sqrt(jnp.sum(e * e, axis=1, keepdims=True))
        en_scr[...] = e / jnp.maximum(nrm, 1e-12)

    zb = z_ref[0]
    big = float(N_CODES)
    nrb = _BM // _RB
    lane = lax.broadcasted_iota(jnp.int32, (_RB, 128), 1).astype(jnp.float32)
    run_m = [jnp.full((_RB, 1), -jnp.inf, jnp.float32) for _ in range(nrb)]
    run_a = [jnp.full((_RB, 1), big, jnp.float32) for _ in range(nrb)]
    for c in range(N_CODES // _BN):
        s = lax.dot_general(
            zb, en_scr[pl.ds(c * _BN, _BN), :],
            (((0,), (1,)), ((), ())),
            preferred_element_type=jnp.float32,
        )
        for r in range(nrb):
            sb = s[r * _RB:(r + 1) * _RB, :]
            v = sb[:, 0:128]
            t_id = jnp.zeros((_RB, 128), jnp.float32)
            for t in range(1, _BN // 128):
                st = sb[:, t * 128:(t + 1) * 128]
                g = st > v
                v = jnp.where(g, st, v)
                t_id = jnp.where(g, jnp.float32(t), t_id)
            colv = t_id * 128.0 + lane
            m = jnp.max(v, axis=1, keepdims=True)
            hit = jnp.where(v == m, colv, big)
            a = jnp.min(hit, axis=1, keepdims=True)
            upd = m > run_m[r]
            run_m[r] = jnp.where(upd, m, run_m[r])
            run_a[r] = jnp.where(upd, a + float(c * _BN), run_a[r])
    out = jnp.concatenate(run_a, axis=0)
    idx_ref[...] = out[:, 0].astype(jnp.int32).reshape(1, 1, _BM)


def _compute_indices(z_chan, emb):
    b = z_chan.shape[0]
    nj = z_chan.shape[2] // _BM
    idx3 = pl.pallas_call(
        _argmax_body,
        grid=(b, nj),
        in_specs=[
            pl.BlockSpec((1, DIM, _BM), lambda i, j: (i, 0, j)),
            pl.BlockSpec((N_CODES, DIM), lambda i, j: (0, 0)),
        ],
        out_specs=pl.BlockSpec((1, 1, _BM), lambda i, j: (i * nj + j, 0, 0)),
        out_shape=jax.ShapeDtypeStruct((b * nj, 1, _BM), jnp.int32),
        scratch_shapes=[pltpu.VMEM((N_CODES, DIM), jnp.float32)],
    )(z_chan, emb)
    return idx3.reshape(-1)


def _sc_gather(emb, idx):
    m = idx.shape[0]
    bpw = m // _SC_WORKERS
    mesh = plsc.VectorSubcoreMesh(core_axis_name="c", subcore_axis_name="s")

    @functools.partial(
        pl.kernel,
        mesh=mesh,
        out_type=jax.ShapeDtypeStruct((m, DIM), jnp.float32),
        compiler_params=pltpu.CompilerParams(use_tc_tiling_on_sc=False),
        scratch_types=[
            pltpu.VMEM((bpw,), jnp.int32),
            pltpu.VMEM((bpw, DIM), jnp.float32),
            pltpu.SemaphoreType.DMA,
        ],
    )
    def gather(table_hbm, idx_hbm, out_hbm, idx_v, rows_v, sem):
        wid = lax.axis_index("s") * 2 + lax.axis_index("c")
        base = wid * bpw
        pltpu.sync_copy(idx_hbm.at[pl.ds(base, bpw)], idx_v)
        copies = [
            pltpu.async_copy(
                table_hbm.at[idx_v.at[pl.ds(j, _GATHER_CHUNK)]],
                rows_v.at[pl.ds(j, _GATHER_CHUNK), :],
                sem,
            )
            for j in range(0, bpw, _GATHER_CHUNK)
        ]
        for cp in copies:
            cp.wait()
        pltpu.sync_copy(rows_v, out_hbm.at[pl.ds(base, bpw)])

    return gather(emb, idx)


def _finalize_body(zq_ref, zqn_ref, bits_ref):
    zq = zq_ref[...]
    qn = jnp.maximum(jnp.sqrt(jnp.sum(zq * zq, axis=1, keepdims=True)), 1e-12)
    zqn = zq / qn
    zqn_ref[...] = zqn
    bits_ref[...] = (zqn * math.sqrt(32.0)).astype(jnp.int32) + 4


def _finalize(zq_raw):
    m = zq_raw.shape[0]
    nb = m // _BF
    return pl.pallas_call(
        _finalize_body,
        grid=(nb,),
        in_specs=[pl.BlockSpec((_BF, DIM), lambda i: (i, 0))],
        out_specs=[
            pl.BlockSpec((_BF, DIM), lambda i: (i, 0)),
            pl.BlockSpec((_BF, DIM), lambda i: (i, 0)),
        ],
        out_shape=[
            jax.ShapeDtypeStruct((m, DIM), jnp.float32),
            jax.ShapeDtypeStruct((m, DIM), jnp.int32),
        ],
    )(zq_raw)


def kernel(z, embedding_weight):
    b, d, h, w = z.shape
    z_chan = z.reshape(b, d, h * w)
    idx = _compute_indices(z_chan, embedding_weight)
    zq_raw = _sc_gather(embedding_weight, idx)
    zqn, bits = _finalize(zq_raw)
    z_q_out = jnp.transpose(zqn.reshape(b, h, w, d), (0, 3, 1, 2))
    bit_indices = bits.reshape(b, h, w, d)
    loss = jnp.zeros((1,), dtype=z.dtype)
    return (z_q_out, idx, bit_indices, loss)
